# baseline (speedup 1.0000x reference)
import functools

import jax
import jax.numpy as jnp
from jax import lax
from jax.experimental import pallas as pl
from jax.experimental.pallas import tpu as pltpu, tpu_sc as plsc


def _node_body(qa_r, x_r, nt_r, ns_r, Wnt_r, bnt_r, Wx2h_r, bx2h_r, Wgat_r,
               asw_r, adw_r, Wge_r, aew_r,
               xl_o, asrc_o, adst_o, m1_o, m2_o, ve_o):
    N = x_r.shape[0]
    bs = qa_r.shape[0]
    npb = N // bs
    rows = lax.broadcasted_iota(jnp.int32, (N, bs), 0)
    cols = lax.broadcasted_iota(jnp.int32, (N, bs), 1)
    sel = (rows == cols * npb).astype(jnp.float32)
    qa_rows = jnp.dot(sel, qa_r[...], preferred_element_type=jnp.float32)
    is_qa = (rows[:, :1] % npb) == 0
    x2 = jnp.where(is_qa, qa_rows, x_r[...])

    Wnt = Wnt_r[...]
    xe = (jnp.dot(nt_r[...], Wnt[0:4, :], preferred_element_type=jnp.float32)
          + ns_r[...] * Wnt[4:5, :] + bnt_r[...])
    Wx2h = Wx2h_r[...]
    h = jnp.maximum(
        jnp.dot(x2, Wx2h[0:128, :], preferred_element_type=jnp.float32)
        + jnp.dot(xe, Wx2h[128:192, :], preferred_element_type=jnp.float32)
        + bx2h_r[...], 0.0)
    xl = jnp.dot(h, Wgat_r[...], preferred_element_type=jnp.float32)
    xl_o[...] = xl
    asrc = jnp.sum(xl * asw_r[...], axis=-1, keepdims=True)
    adst = jnp.sum(xl * adw_r[...], axis=-1, keepdims=True)
    asrc_o[...] = asrc
    adst_o[...] = adst
    m1_o[...] = jnp.max(asrc).reshape(1, 1)
    m2_o[...] = jnp.max(adst).reshape(1, 1)
    ve_o[...] = lax.dot_general(aew_r[...], Wge_r[...],
                                (((1,), (1,)), ((), ())),
                                preferred_element_type=jnp.float32)


def _node_prologue(qa, x, nt, ns, Wnt, bnt, Wx2h, bx2h, Wgat, asw, adw, Wge, aew):
    N = x.shape[0]
    return pl.pallas_call(
        _node_body,
        out_shape=(
            jax.ShapeDtypeStruct((N, 128), jnp.float32),
            jax.ShapeDtypeStruct((N, 1), jnp.float32),
            jax.ShapeDtypeStruct((N, 1), jnp.float32),
            jax.ShapeDtypeStruct((1, 1), jnp.float32),
            jax.ShapeDtypeStruct((1, 1), jnp.float32),
            jax.ShapeDtypeStruct((1, 128), jnp.float32),
        ),
    )(qa, x, nt, ns, Wnt, bnt, Wx2h, bx2h, Wgat, asw, adw, Wge, aew)


def _edge_body(ea_r, We1_r, be1_r, We2_r, be2_r, ve_r, aedge_o, m3_o, acc):
    i = pl.program_id(0)
    n = pl.num_programs(0)
    t = jnp.maximum(jnp.dot(ea_r[...], We1_r[...],
                            preferred_element_type=jnp.float32) + be1_r[...], 0.0)
    t = jnp.maximum(jnp.dot(t, We2_r[...],
                            preferred_element_type=jnp.float32) + be2_r[...], 0.0)
    a = lax.dot_general(t, ve_r[...], (((1,), (1,)), ((), ())),
                        preferred_element_type=jnp.float32)
    aedge_o[...] = a

    @pl.when(i == 0)
    def _():
        acc[0, 0] = -jnp.inf

    acc[0, 0] = jnp.maximum(acc[0, 0], jnp.max(a))

    @pl.when(i == n - 1)
    def _():
        m3_o[...] = jnp.full((1, 1), acc[0, 0], jnp.float32)


def _edge_mlp(edge_attr, We1, be1, We2, be2, ve):
    E, e_in = edge_attr.shape
    T = 16000
    grid = (E // T,)
    return pl.pallas_call(
        _edge_body,
        grid=grid,
        in_specs=[
            pl.BlockSpec((T, e_in), lambda i: (i, 0)),
            pl.BlockSpec((e_in, 128), lambda i: (0, 0)),
            pl.BlockSpec((1, 128), lambda i: (0, 0)),
            pl.BlockSpec((128, 128), lambda i: (0, 0)),
            pl.BlockSpec((1, 128), lambda i: (0, 0)),
            pl.BlockSpec((1, 128), lambda i: (0, 0)),
        ],
        out_specs=(
            pl.BlockSpec((T, 1), lambda i: (i, 0)),
            pl.BlockSpec((1, 1), lambda i: (0, 0)),
        ),
        out_shape=(
            jax.ShapeDtypeStruct((E, 1), jnp.float32),
            jax.ShapeDtypeStruct((1, 1), jnp.float32),
        ),
        scratch_shapes=[pltpu.SMEM((1, 1), jnp.float32)],
    )(edge_attr, We1, be1, We2, be2, ve)


_B = 80
_NSUB = 16
_NW = 32


_NSC = 5


def _sc_body(src4_r, dst4_r, aef_r, asrc_r, adst_r, xl_r, shift_r, zr_r, zd_r,
             acc_o, den_o,
             srci2, dsti2, aef, asb, adb, exb, rows, shiftv,
             acc_sh, den_sh, sem0, sem1):
    E = aef_r.shape[0]
    epw = E // _NW
    nbw = epw // _B
    nbs = nbw // _NSC
    eps = epw // _NSC
    npad = den_sh.shape[0]
    rpt = npad // _NSUB
    cid = lax.axis_index("c")
    sid = lax.axis_index("s")
    wid = cid * _NSUB + sid
    sems = (sem0, sem1)

    pltpu.sync_copy(zr_r.at[pl.ds(sid * rpt, rpt)],
                    acc_sh.at[pl.ds(sid * rpt, rpt)])

    @pl.when(sid == 0)
    def _():
        pltpu.sync_copy(zd_r, den_sh)

    pltpu.sync_copy(shift_r, shiftv)
    plsc.subcore_barrier()

    def issue(j, b):
        pltpu.async_copy(xl_r.at[srci2.at[j]], rows.at[b], sems[b])
        pltpu.async_copy(asrc_r.at[srci2.at[j]], asb.at[b], sems[b])
        pltpu.async_copy(adst_r.at[dsti2.at[j]], adb.at[b], sems[b])

    def drain(j, b):
        pltpu.make_async_copy(xl_r.at[srci2.at[j]], rows.at[b],
                              sems[b]).wait()
        pltpu.make_async_copy(asrc_r.at[srci2.at[j]], asb.at[b],
                              sems[b]).wait()
        pltpu.make_async_copy(adst_r.at[dsti2.at[j]], adb.at[b],
                              sems[b]).wait()

    def process(j, b):
        shv = shiftv[...]
        for i in range(_B // 16):
            al = (asb[b, pl.ds(i * 16, 16)] + adb[b, pl.ds(i * 16, 16)]
                  + aef[pl.ds(j * _B + i * 16, 16)])
            al = jnp.where(al > 0.0, al, al * 0.2)
            exb[b, pl.ds(i * 16, 16)] = jnp.exp(al - shv)

        def rowgrp(i, c):
            ev = exb[b, pl.ds(i * 16, 16)]
            for r in range(16):
                e = ev[r]
                row = i * 16 + r
                for k in range(8):
                    rows[b, row, pl.ds(k * 16, 16)] = (
                        rows[b, row, pl.ds(k * 16, 16)] * e)
            return c

        lax.fori_loop(0, _B // 16, rowgrp, 0, unroll=False)

        pltpu.sync_copy(rows.at[b], acc_sh.at[dsti2.at[j]], add=True)
        pltpu.sync_copy(exb.at[b], den_sh.at[dsti2.at[j]], add=True)

    def superchunk(s, carry0):
        pltpu.sync_copy(src4_r.at[wid, s], srci2)
        pltpu.sync_copy(dst4_r.at[wid, s], dsti2)
        pltpu.sync_copy(aef_r.at[pl.ds(wid * epw + s * eps, eps)], aef)

        issue(0, 0)
        issue(1, 1)

        def pair(g, carry):
            for b in range(2):
                j = 2 * g + b
                drain(j, b)
                process(j, b)

                @pl.when(j + 2 < nbs)
                def _():
                    issue(j + 2, b)
            return carry

        lax.fori_loop(0, (nbs - 1) // 2, pair, 0, unroll=False)
        jt = nbs - 1
        drain(jt, 0)
        process(jt, 0)
        return carry0

    lax.fori_loop(0, _NSC, superchunk, 0, unroll=False)
    plsc.subcore_barrier()

    pltpu.sync_copy(acc_sh.at[pl.ds(sid * rpt, rpt)],
                    acc_o.at[pl.ds(cid * npad + sid * rpt, rpt)])

    @pl.when(sid == 0)
    def _():
        pltpu.sync_copy(den_sh, den_o.at[pl.ds(cid * npad, npad)])


def _sc_message_passing(src, dst, a_edge, a_src, a_dst, xl, shift):
    N = xl.shape[0]
    E = src.shape[0]
    epw = E // _NW
    nbw = epw // _B
    nbs = nbw // _NSC
    npad = ((N + 8 * _NSUB - 1) // (8 * _NSUB)) * (8 * _NSUB)
    mesh = plsc.VectorSubcoreMesh(core_axis_name="c", subcore_axis_name="s")
    src4 = src.reshape(_NW, _NSC, nbs, _B)
    dst4 = dst.reshape(_NW, _NSC, nbs, _B)

    kfn = functools.partial(
        pl.kernel,
        out_type=(
            jax.ShapeDtypeStruct((2 * npad, 128), jnp.float32),
            jax.ShapeDtypeStruct((2 * npad,), jnp.float32),
        ),
        mesh=mesh,
        scratch_types=[
            pltpu.VMEM((nbs, _B), jnp.int32),
            pltpu.VMEM((nbs, _B), jnp.int32),
            pltpu.VMEM((epw // _NSC,), jnp.float32),
            pltpu.VMEM((2, _B), jnp.float32),
            pltpu.VMEM((2, _B), jnp.float32),
            pltpu.VMEM((2, _B), jnp.float32),
            pltpu.VMEM((2, _B, 128), jnp.float32),
            pltpu.VMEM((16,), jnp.float32),
            pltpu.VMEM_SHARED((npad, 128), jnp.float32),
            pltpu.VMEM_SHARED((npad,), jnp.float32),
            pltpu.SemaphoreType.DMA,
            pltpu.SemaphoreType.DMA,
        ],
        compiler_params=pltpu.CompilerParams(needs_layout_passes=False),
    )(_sc_body)
    zr = jnp.zeros((npad, 128), jnp.float32)
    zd = jnp.zeros((npad,), jnp.float32)
    accp, denp = kfn(src4, dst4, a_edge, a_src, a_dst, xl, shift, zr, zd)
    return (accp, denp), npad


def _fin_body(accp_r, denp_r, n2g_r, bgat_r, h0_o, pooled_o):
    N = n2g_r.shape[0]
    npad = accp_r.shape[0] // 2
    bs = h0_o.shape[0]
    npb = N // bs
    acc = accp_r[0:N, :] + accp_r[npad:npad + N, :]
    den = denp_r[0:N, :] + denp_r[npad:npad + N, :]
    out = acc / (den + 1e-16) + bgat_r[...]

    rows = lax.broadcasted_iota(jnp.int32, (N, bs), 0)
    cols = lax.broadcasted_iota(jnp.int32, (N, bs), 1)
    sel = (rows == cols * npb).astype(jnp.float32)
    h0_o[...] = lax.dot_general(sel, out, (((0,), (0,)), ((), ())),
                                preferred_element_type=jnp.float32)

    g = (n2g_r[...] == cols).astype(jnp.float32)
    psum = lax.dot_general(g, out, (((0,), (0,)), ((), ())),
                           preferred_element_type=jnp.float32)
    ones = jnp.ones((N, 1), jnp.float32)
    cnt = lax.dot_general(g, ones, (((0,), (0,)), ((), ())),
                          preferred_element_type=jnp.float32)
    pooled_o[...] = psum / jnp.maximum(cnt, 1.0)


def _finisher(accp, denp, n2g, bgat, bs):
    N = n2g.shape[0]
    return pl.pallas_call(
        _fin_body,
        out_shape=(
            jax.ShapeDtypeStruct((bs, 128), jnp.float32),
            jax.ShapeDtypeStruct((bs, 128), jnp.float32),
        ),
    )(accp, denp, n2g, bgat)


def kernel(qa_emb, x, node_ids, node_types, node_scores, edge_index, edge_type,
           edge_attr, node2graph, W_nt, b_nt, W_x2h, b_x2h, W_e1, b_e1, W_e2,
           b_e2, W_gat, att_src, att_dst, W_ge, att_edge, b_gat):
    bs = qa_emb.shape[0]

    xl, a_src, a_dst, m1, m2, ve = _node_prologue(
        qa_emb, x, node_types, node_scores, W_nt, b_nt.reshape(1, -1),
        W_x2h, b_x2h.reshape(1, -1), W_gat, att_src.reshape(1, -1),
        att_dst.reshape(1, -1), W_ge, att_edge.reshape(1, -1))

    a_edge, m3 = _edge_mlp(edge_attr, W_e1, b_e1.reshape(1, -1), W_e2,
                           b_e2.reshape(1, -1), ve)

    shift = jnp.full((16,), m1[0, 0] + m2[0, 0] + m3[0, 0], jnp.float32)
    src = edge_index[0].astype(jnp.int32)
    dst = edge_index[1].astype(jnp.int32)
    (accp, denp), _npad = _sc_message_passing(
        src, dst, a_edge.reshape(-1), a_src.reshape(-1), a_dst.reshape(-1),
        xl, shift)

    h0, pooled = _finisher(accp, denp.reshape(-1, 1),
                           node2graph.astype(jnp.int32).reshape(-1, 1),
                           b_gat.reshape(1, -1), bs)
    return (h0, pooled)


# speedup vs baseline: 1.6118x; 1.6118x over previous
import functools

import jax
import jax.numpy as jnp
from jax import lax
from jax.experimental import pallas as pl
from jax.experimental.pallas import tpu as pltpu, tpu_sc as plsc


def _node_body(qa_r, x_r, nt_r, ns_r, Wnt_r, bnt_r, Wx2h_r, bx2h_r, Wgat_r,
               asw_r, adw_r, Wge_r, aew_r,
               xl_o, asrc_o, adst_o, m1_o, m2_o, ve_o):
    N = x_r.shape[0]
    bs = qa_r.shape[0]
    npb = N // bs
    rows = lax.broadcasted_iota(jnp.int32, (N, bs), 0)
    cols = lax.broadcasted_iota(jnp.int32, (N, bs), 1)
    sel = (rows == cols * npb).astype(jnp.float32)
    qa_rows = jnp.dot(sel, qa_r[...], preferred_element_type=jnp.float32)
    is_qa = (rows[:, :1] % npb) == 0
    x2 = jnp.where(is_qa, qa_rows, x_r[...])

    Wnt = Wnt_r[...]
    xe = (jnp.dot(nt_r[...], Wnt[0:4, :], preferred_element_type=jnp.float32)
          + ns_r[...] * Wnt[4:5, :] + bnt_r[...])
    Wx2h = Wx2h_r[...]
    h = jnp.maximum(
        jnp.dot(x2, Wx2h[0:128, :], preferred_element_type=jnp.float32)
        + jnp.dot(xe, Wx2h[128:192, :], preferred_element_type=jnp.float32)
        + bx2h_r[...], 0.0)
    xl = jnp.dot(h, Wgat_r[...], preferred_element_type=jnp.float32)
    xl_o[...] = xl
    asrc = lax.dot_general(asw_r[...], xl, (((1,), (1,)), ((), ())),
                           preferred_element_type=jnp.float32)
    adst = lax.dot_general(adw_r[...], xl, (((1,), (1,)), ((), ())),
                           preferred_element_type=jnp.float32)
    asrc_o[...] = asrc
    adst_o[...] = adst
    m1_o[...] = jnp.max(asrc).reshape(1, 1)
    m2_o[...] = jnp.max(adst).reshape(1, 1)
    ve_o[...] = lax.dot_general(aew_r[...], Wge_r[...],
                                (((1,), (1,)), ((), ())),
                                preferred_element_type=jnp.float32)


def _node_prologue(qa, x, nt, ns, Wnt, bnt, Wx2h, bx2h, Wgat, asw, adw, Wge, aew):
    N = x.shape[0]
    return pl.pallas_call(
        _node_body,
        out_shape=(
            jax.ShapeDtypeStruct((N, 128), jnp.float32),
            jax.ShapeDtypeStruct((1, N), jnp.float32),
            jax.ShapeDtypeStruct((1, N), jnp.float32),
            jax.ShapeDtypeStruct((1, 1), jnp.float32),
            jax.ShapeDtypeStruct((1, 1), jnp.float32),
            jax.ShapeDtypeStruct((1, 128), jnp.float32),
        ),
    )(qa, x, nt, ns, Wnt, bnt, Wx2h, bx2h, Wgat, asw, adw, Wge, aew)


def _edge_body(ea_r, We1_r, be1_r, We2_r, be2_r, ve_r, aedge_o, m3_o, acc):
    i = pl.program_id(0)
    n = pl.num_programs(0)
    t = jnp.maximum(
        lax.dot_general(We1_r[...], ea_r[...], (((0,), (0,)), ((), ())),
                        preferred_element_type=jnp.float32) + be1_r[...], 0.0)
    t = jnp.maximum(
        lax.dot_general(We2_r[...], t, (((0,), (0,)), ((), ())),
                        preferred_element_type=jnp.float32) + be2_r[...], 0.0)
    a = lax.dot_general(ve_r[...], t, (((1,), (0,)), ((), ())),
                        preferred_element_type=jnp.float32)
    aedge_o[0] = a

    @pl.when(i == 0)
    def _():
        acc[0, 0] = -jnp.inf

    acc[0, 0] = jnp.maximum(acc[0, 0], jnp.max(a))

    @pl.when(i == n - 1)
    def _():
        m3_o[...] = jnp.full((1, 1), acc[0, 0], jnp.float32)


def _edge_mlp(edge_attr_t, We1, be1, We2, be2, ve):
    e_in, E = edge_attr_t.shape
    T = 16000
    n = E // T
    return pl.pallas_call(
        _edge_body,
        grid=(n,),
        in_specs=[
            pl.BlockSpec((e_in, T), lambda i: (0, i)),
            pl.BlockSpec((e_in, 128), lambda i: (0, 0)),
            pl.BlockSpec((128, 1), lambda i: (0, 0)),
            pl.BlockSpec((128, 128), lambda i: (0, 0)),
            pl.BlockSpec((128, 1), lambda i: (0, 0)),
            pl.BlockSpec((1, 128), lambda i: (0, 0)),
        ],
        out_specs=(
            pl.BlockSpec((1, 1, T), lambda i: (i, 0, 0)),
            pl.BlockSpec((1, 1), lambda i: (0, 0)),
        ),
        out_shape=(
            jax.ShapeDtypeStruct((n, 1, T), jnp.float32),
            jax.ShapeDtypeStruct((1, 1), jnp.float32),
        ),
        scratch_shapes=[pltpu.SMEM((1, 1), jnp.float32)],
    )(edge_attr_t, We1, be1, We2, be2, ve)


_B = 80
_NSUB = 16
_NW = 32


_NSC = 5


def _sc_body(src4_r, dst4_r, aef_r, asrc_r, adst_r, xl_r, shift_r, zr_r, zd_r,
             acc_o, den_o,
             srci2, dsti2, aef, asb, adb, exb, rows, shiftv,
             acc_sh, den_sh, sem0, sem1):
    E = aef_r.shape[0]
    epw = E // _NW
    nbw = epw // _B
    nbs = nbw // _NSC
    eps = epw // _NSC
    npad = den_sh.shape[0]
    rpt = npad // _NSUB
    cid = lax.axis_index("c")
    sid = lax.axis_index("s")
    wid = cid * _NSUB + sid
    sems = (sem0, sem1)

    pltpu.sync_copy(zr_r.at[pl.ds(sid * rpt, rpt)],
                    acc_sh.at[pl.ds(sid * rpt, rpt)])

    @pl.when(sid == 0)
    def _():
        pltpu.sync_copy(zd_r, den_sh)

    pltpu.sync_copy(shift_r, shiftv)
    plsc.subcore_barrier()

    def issue(j, b):
        pltpu.async_copy(xl_r.at[srci2.at[j]], rows.at[b], sems[b])
        pltpu.async_copy(asrc_r.at[srci2.at[j]], asb.at[b], sems[b])
        pltpu.async_copy(adst_r.at[dsti2.at[j]], adb.at[b], sems[b])

    def drain(j, b):
        pltpu.make_async_copy(xl_r.at[srci2.at[j]], rows.at[b],
                              sems[b]).wait()
        pltpu.make_async_copy(asrc_r.at[srci2.at[j]], asb.at[b],
                              sems[b]).wait()
        pltpu.make_async_copy(adst_r.at[dsti2.at[j]], adb.at[b],
                              sems[b]).wait()

    def process(j, b):
        shv = shiftv[...]
        for i in range(_B // 16):
            al = (asb[b, pl.ds(i * 16, 16)] + adb[b, pl.ds(i * 16, 16)]
                  + aef[pl.ds(j * _B + i * 16, 16)])
            al = jnp.where(al > 0.0, al, al * 0.2)
            exb[b, pl.ds(i * 16, 16)] = jnp.exp(al - shv)

        def rowgrp(i, c):
            ev = exb[b, pl.ds(i * 16, 16)]
            for r in range(16):
                e = ev[r]
                row = i * 16 + r
                for k in range(8):
                    rows[b, row, pl.ds(k * 16, 16)] = (
                        rows[b, row, pl.ds(k * 16, 16)] * e)
            return c

        lax.fori_loop(0, _B // 16, rowgrp, 0, unroll=False)

        pltpu.sync_copy(rows.at[b], acc_sh.at[dsti2.at[j]], add=True)
        pltpu.sync_copy(exb.at[b], den_sh.at[dsti2.at[j]], add=True)

    def superchunk(s, carry0):
        pltpu.sync_copy(src4_r.at[wid, s], srci2)
        pltpu.sync_copy(dst4_r.at[wid, s], dsti2)
        pltpu.sync_copy(aef_r.at[pl.ds(wid * epw + s * eps, eps)], aef)

        issue(0, 0)
        issue(1, 1)

        def pair(g, carry):
            for b in range(2):
                j = 2 * g + b
                drain(j, b)
                process(j, b)

                @pl.when(j + 2 < nbs)
                def _():
                    issue(j + 2, b)
            return carry

        lax.fori_loop(0, (nbs - 1) // 2, pair, 0, unroll=False)
        jt = nbs - 1
        drain(jt, 0)
        process(jt, 0)
        return carry0

    lax.fori_loop(0, _NSC, superchunk, 0, unroll=False)
    plsc.subcore_barrier()

    pltpu.sync_copy(acc_sh.at[pl.ds(sid * rpt, rpt)],
                    acc_o.at[pl.ds(cid * npad + sid * rpt, rpt)])

    @pl.when(sid == 0)
    def _():
        pltpu.sync_copy(den_sh, den_o.at[pl.ds(cid * npad, npad)])


def _sc_message_passing(src, dst, a_edge, a_src, a_dst, xl, shift):
    N = xl.shape[0]
    E = src.shape[0]
    epw = E // _NW
    nbw = epw // _B
    nbs = nbw // _NSC
    npad = ((N + 8 * _NSUB - 1) // (8 * _NSUB)) * (8 * _NSUB)
    mesh = plsc.VectorSubcoreMesh(core_axis_name="c", subcore_axis_name="s")
    src4 = src.reshape(_NW, _NSC, nbs, _B)
    dst4 = dst.reshape(_NW, _NSC, nbs, _B)

    kfn = functools.partial(
        pl.kernel,
        out_type=(
            jax.ShapeDtypeStruct((2 * npad, 128), jnp.float32),
            jax.ShapeDtypeStruct((2 * npad,), jnp.float32),
        ),
        mesh=mesh,
        scratch_types=[
            pltpu.VMEM((nbs, _B), jnp.int32),
            pltpu.VMEM((nbs, _B), jnp.int32),
            pltpu.VMEM((epw // _NSC,), jnp.float32),
            pltpu.VMEM((2, _B), jnp.float32),
            pltpu.VMEM((2, _B), jnp.float32),
            pltpu.VMEM((2, _B), jnp.float32),
            pltpu.VMEM((2, _B, 128), jnp.float32),
            pltpu.VMEM((16,), jnp.float32),
            pltpu.VMEM_SHARED((npad, 128), jnp.float32),
            pltpu.VMEM_SHARED((npad,), jnp.float32),
            pltpu.SemaphoreType.DMA,
            pltpu.SemaphoreType.DMA,
        ],
        compiler_params=pltpu.CompilerParams(needs_layout_passes=False),
    )(_sc_body)
    zr = jnp.zeros((npad, 128), jnp.float32)
    zd = jnp.zeros((npad,), jnp.float32)
    accp, denp = kfn(src4, dst4, a_edge, a_src, a_dst, xl, shift, zr, zd)
    return (accp, denp), npad


def _fin_body(accp_r, denp_r, n2g_r, bgat_r, h0_o, pooled_o):
    N = n2g_r.shape[0]
    npad = accp_r.shape[0] // 2
    bs = h0_o.shape[0]
    npb = N // bs
    acc = accp_r[0:N, :] + accp_r[npad:npad + N, :]
    den = denp_r[0:N, :] + denp_r[npad:npad + N, :]
    out = acc / (den + 1e-16) + bgat_r[...]

    rows = lax.broadcasted_iota(jnp.int32, (N, bs), 0)
    cols = lax.broadcasted_iota(jnp.int32, (N, bs), 1)
    sel = (rows == cols * npb).astype(jnp.float32)
    h0_o[...] = lax.dot_general(sel, out, (((0,), (0,)), ((), ())),
                                preferred_element_type=jnp.float32)

    g = (n2g_r[...] == cols).astype(jnp.float32)
    psum = lax.dot_general(g, out, (((0,), (0,)), ((), ())),
                           preferred_element_type=jnp.float32)
    ones = jnp.ones((N, 1), jnp.float32)
    cnt = lax.dot_general(g, ones, (((0,), (0,)), ((), ())),
                          preferred_element_type=jnp.float32)
    pooled_o[...] = psum / jnp.maximum(cnt, 1.0)


def _finisher(accp, denp, n2g, bgat, bs):
    N = n2g.shape[0]
    return pl.pallas_call(
        _fin_body,
        out_shape=(
            jax.ShapeDtypeStruct((bs, 128), jnp.float32),
            jax.ShapeDtypeStruct((bs, 128), jnp.float32),
        ),
    )(accp, denp, n2g, bgat)


def kernel(qa_emb, x, node_ids, node_types, node_scores, edge_index, edge_type,
           edge_attr, node2graph, W_nt, b_nt, W_x2h, b_x2h, W_e1, b_e1, W_e2,
           b_e2, W_gat, att_src, att_dst, W_ge, att_edge, b_gat):
    bs = qa_emb.shape[0]

    xl, a_src, a_dst, m1, m2, ve = _node_prologue(
        qa_emb, x, node_types, node_scores, W_nt, b_nt.reshape(1, -1),
        W_x2h, b_x2h.reshape(1, -1), W_gat, att_src.reshape(1, -1),
        att_dst.reshape(1, -1), W_ge, att_edge.reshape(1, -1))

    a_edge, m3 = _edge_mlp(edge_attr.T, W_e1, b_e1.reshape(-1, 1), W_e2,
                           b_e2.reshape(-1, 1), ve)

    shift = jnp.full((16,), m1[0, 0] + m2[0, 0] + m3[0, 0], jnp.float32)
    src = edge_index[0].astype(jnp.int32)
    dst = edge_index[1].astype(jnp.int32)
    (accp, denp), _npad = _sc_message_passing(
        src, dst, a_edge.reshape(-1), a_src.reshape(-1), a_dst.reshape(-1),
        xl, shift)

    h0, pooled = _finisher(accp, denp.reshape(-1, 1),
                           node2graph.astype(jnp.int32).reshape(-1, 1),
                           b_gat.reshape(1, -1), bs)
    return (h0, pooled)


# speedup vs baseline: 1.6656x; 1.0334x over previous
import functools

import jax
import jax.numpy as jnp
from jax import lax
from jax.experimental import pallas as pl
from jax.experimental.pallas import tpu as pltpu, tpu_sc as plsc


def _node_body(qa_r, x_r, nt_r, ns_r, Wnt_r, bnt_r, Wx2h_r, bx2h_r, Wgat_r,
               asw_r, adw_r, Wge_r, aew_r,
               xl_o, asrc_o, adst_o, m1_o, m2_o, ve_o):
    N = x_r.shape[0]
    bs = qa_r.shape[0]
    npb = N // bs
    rows = lax.broadcasted_iota(jnp.int32, (N, bs), 0)
    cols = lax.broadcasted_iota(jnp.int32, (N, bs), 1)
    sel = (rows == cols * npb).astype(jnp.float32)
    qa_rows = jnp.dot(sel, qa_r[...], preferred_element_type=jnp.float32)
    is_qa = (rows[:, :1] % npb) == 0
    x2 = jnp.where(is_qa, qa_rows, x_r[...])

    Wnt = Wnt_r[...]
    xe = (jnp.dot(nt_r[...], Wnt[0:4, :], preferred_element_type=jnp.float32)
          + ns_r[...] * Wnt[4:5, :] + bnt_r[...])
    Wx2h = Wx2h_r[...]
    h = jnp.maximum(
        jnp.dot(x2, Wx2h[0:128, :], preferred_element_type=jnp.float32)
        + jnp.dot(xe, Wx2h[128:192, :], preferred_element_type=jnp.float32)
        + bx2h_r[...], 0.0)
    xl = jnp.dot(h, Wgat_r[...], preferred_element_type=jnp.float32)
    xl_o[...] = xl
    asrc = lax.dot_general(asw_r[...], xl, (((1,), (1,)), ((), ())),
                           preferred_element_type=jnp.float32)
    adst = lax.dot_general(adw_r[...], xl, (((1,), (1,)), ((), ())),
                           preferred_element_type=jnp.float32)
    asrc_o[...] = asrc
    adst_o[...] = adst
    m1_o[...] = jnp.max(asrc).reshape(1, 1)
    m2_o[...] = jnp.max(adst).reshape(1, 1)
    ve_o[...] = lax.dot_general(aew_r[...], Wge_r[...],
                                (((1,), (1,)), ((), ())),
                                preferred_element_type=jnp.float32)


def _node_prologue(qa, x, nt, ns, Wnt, bnt, Wx2h, bx2h, Wgat, asw, adw, Wge, aew):
    N = x.shape[0]
    return pl.pallas_call(
        _node_body,
        out_shape=(
            jax.ShapeDtypeStruct((N, 128), jnp.float32),
            jax.ShapeDtypeStruct((1, N), jnp.float32),
            jax.ShapeDtypeStruct((1, N), jnp.float32),
            jax.ShapeDtypeStruct((1, 1), jnp.float32),
            jax.ShapeDtypeStruct((1, 1), jnp.float32),
            jax.ShapeDtypeStruct((1, 128), jnp.float32),
        ),
    )(qa, x, nt, ns, Wnt, bnt, Wx2h, bx2h, Wgat, asw, adw, Wge, aew)


def _edge_body(ea_r, We1_r, be1_r, We2_r, be2_r, ve_r, aedge_o, m3_o, acc):
    i = pl.program_id(0)
    n = pl.num_programs(0)
    t = jnp.maximum(
        lax.dot_general(We1_r[...], ea_r[...], (((0,), (0,)), ((), ())),
                        preferred_element_type=jnp.float32) + be1_r[...], 0.0)
    t = jnp.maximum(
        lax.dot_general(We2_r[...], t, (((0,), (0,)), ((), ())),
                        preferred_element_type=jnp.float32) + be2_r[...], 0.0)
    a = lax.dot_general(ve_r[...], t, (((1,), (0,)), ((), ())),
                        preferred_element_type=jnp.float32)
    aedge_o[0] = a

    @pl.when(i == 0)
    def _():
        acc[0, 0] = -jnp.inf

    acc[0, 0] = jnp.maximum(acc[0, 0], jnp.max(a))

    @pl.when(i == n - 1)
    def _():
        m3_o[...] = jnp.full((1, 1), acc[0, 0], jnp.float32)


def _edge_mlp(edge_attr_t, We1, be1, We2, be2, ve):
    e_in, E = edge_attr_t.shape
    T = 16000
    n = E // T
    return pl.pallas_call(
        _edge_body,
        grid=(n,),
        in_specs=[
            pl.BlockSpec((e_in, T), lambda i: (0, i)),
            pl.BlockSpec((e_in, 128), lambda i: (0, 0)),
            pl.BlockSpec((128, 1), lambda i: (0, 0)),
            pl.BlockSpec((128, 128), lambda i: (0, 0)),
            pl.BlockSpec((128, 1), lambda i: (0, 0)),
            pl.BlockSpec((1, 128), lambda i: (0, 0)),
        ],
        out_specs=(
            pl.BlockSpec((1, 1, T), lambda i: (i, 0, 0)),
            pl.BlockSpec((1, 1), lambda i: (0, 0)),
        ),
        out_shape=(
            jax.ShapeDtypeStruct((n, 1, T), jnp.float32),
            jax.ShapeDtypeStruct((1, 1), jnp.float32),
        ),
        scratch_shapes=[pltpu.SMEM((1, 1), jnp.float32)],
    )(edge_attr_t, We1, be1, We2, be2, ve)


_B = 80
_NSUB = 16
_NW = 32


_NSC = 5


def _sc_body(ei5_r, aef_r, asrc_r, adst_r, xl_r, shift_r, zr_r, zd_r,
             acc_o, den_o,
             srci2, dsti2, aef, asb, adb, exb, rows, shiftv,
             acc_sh, den_sh, sem0, sem1):
    E = aef_r.shape[0]
    epw = E // _NW
    nbw = epw // _B
    nbs = nbw // _NSC
    eps = epw // _NSC
    npad = den_sh.shape[0]
    rpt = npad // _NSUB
    cid = lax.axis_index("c")
    sid = lax.axis_index("s")
    wid = cid * _NSUB + sid
    sems = (sem0, sem1)

    pltpu.sync_copy(zr_r.at[pl.ds(sid * rpt, rpt)],
                    acc_sh.at[pl.ds(sid * rpt, rpt)])

    @pl.when(sid == 0)
    def _():
        pltpu.sync_copy(zd_r, den_sh)

    pltpu.sync_copy(shift_r, shiftv)
    plsc.subcore_barrier()

    def issue(j, b):
        pltpu.async_copy(xl_r.at[srci2.at[j]], rows.at[b], sems[b])
        pltpu.async_copy(asrc_r.at[srci2.at[j]], asb.at[b], sems[b])
        pltpu.async_copy(adst_r.at[dsti2.at[j]], adb.at[b], sems[b])

    def drain(j, b):
        pltpu.make_async_copy(xl_r.at[srci2.at[j]], rows.at[b],
                              sems[b]).wait()
        pltpu.make_async_copy(asrc_r.at[srci2.at[j]], asb.at[b],
                              sems[b]).wait()
        pltpu.make_async_copy(adst_r.at[dsti2.at[j]], adb.at[b],
                              sems[b]).wait()

    def process(j, b):
        shv = shiftv[...]
        for i in range(_B // 16):
            al = (asb[b, pl.ds(i * 16, 16)] + adb[b, pl.ds(i * 16, 16)]
                  + aef[pl.ds(j * _B + i * 16, 16)])
            al = jnp.where(al > 0.0, al, al * 0.2)
            exb[b, pl.ds(i * 16, 16)] = jnp.exp(al - shv)

        def rowgrp(i, c):
            ev = exb[b, pl.ds(i * 16, 16)]
            for r in range(16):
                e = ev[r]
                row = i * 16 + r
                for k in range(8):
                    rows[b, row, pl.ds(k * 16, 16)] = (
                        rows[b, row, pl.ds(k * 16, 16)] * e)
            return c

        lax.fori_loop(0, _B // 16, rowgrp, 0, unroll=False)

        pltpu.sync_copy(rows.at[b], acc_sh.at[dsti2.at[j]], add=True)
        pltpu.sync_copy(exb.at[b], den_sh.at[dsti2.at[j]], add=True)

    def superchunk(s, carry0):
        pltpu.sync_copy(ei5_r.at[0, wid, s], srci2)
        pltpu.sync_copy(ei5_r.at[1, wid, s], dsti2)
        pltpu.sync_copy(aef_r.at[pl.ds(wid * epw + s * eps, eps)], aef)

        issue(0, 0)
        issue(1, 1)

        def pair(g, carry):
            for b in range(2):
                j = 2 * g + b
                drain(j, b)
                process(j, b)

                @pl.when(j + 2 < nbs)
                def _():
                    issue(j + 2, b)
            return carry

        lax.fori_loop(0, (nbs - 1) // 2, pair, 0, unroll=False)
        jt = nbs - 1
        drain(jt, 0)
        process(jt, 0)
        return carry0

    lax.fori_loop(0, _NSC, superchunk, 0, unroll=False)
    plsc.subcore_barrier()

    pltpu.sync_copy(acc_sh.at[pl.ds(sid * rpt, rpt)],
                    acc_o.at[pl.ds(cid * npad + sid * rpt, rpt)])

    @pl.when(sid == 0)
    def _():
        pltpu.sync_copy(den_sh, den_o.at[pl.ds(cid * npad, npad)])


def _sc_message_passing(edge_index, a_edge, a_src, a_dst, xl, shift):
    N = xl.shape[0]
    E = edge_index.shape[1]
    epw = E // _NW
    nbw = epw // _B
    nbs = nbw // _NSC
    npad = ((N + 8 * _NSUB - 1) // (8 * _NSUB)) * (8 * _NSUB)
    mesh = plsc.VectorSubcoreMesh(core_axis_name="c", subcore_axis_name="s")
    ei5 = edge_index.reshape(2, _NW, _NSC, nbs, _B)

    kfn = functools.partial(
        pl.kernel,
        out_type=(
            jax.ShapeDtypeStruct((2 * npad, 128), jnp.float32),
            jax.ShapeDtypeStruct((2 * npad,), jnp.float32),
        ),
        mesh=mesh,
        scratch_types=[
            pltpu.VMEM((nbs, _B), jnp.int32),
            pltpu.VMEM((nbs, _B), jnp.int32),
            pltpu.VMEM((epw // _NSC,), jnp.float32),
            pltpu.VMEM((2, _B), jnp.float32),
            pltpu.VMEM((2, _B), jnp.float32),
            pltpu.VMEM((2, _B), jnp.float32),
            pltpu.VMEM((2, _B, 128), jnp.float32),
            pltpu.VMEM((16,), jnp.float32),
            pltpu.VMEM_SHARED((npad, 128), jnp.float32),
            pltpu.VMEM_SHARED((npad,), jnp.float32),
            pltpu.SemaphoreType.DMA,
            pltpu.SemaphoreType.DMA,
        ],
        compiler_params=pltpu.CompilerParams(needs_layout_passes=False),
    )(_sc_body)
    zr = jnp.zeros((npad, 128), jnp.float32)
    zd = jnp.zeros((npad,), jnp.float32)
    accp, denp = kfn(ei5, a_edge, a_src, a_dst, xl, shift, zr, zd)
    return (accp, denp), npad


def _fin_body(accp_r, denp_r, n2g_r, bgat_r, h0_o, pooled_o):
    N = n2g_r.shape[0]
    npad = accp_r.shape[0] // 2
    bs = h0_o.shape[0]
    npb = N // bs
    acc = accp_r[0:N, :] + accp_r[npad:npad + N, :]
    den = denp_r[0:N, :] + denp_r[npad:npad + N, :]
    out = acc / (den + 1e-16) + bgat_r[...]

    rows = lax.broadcasted_iota(jnp.int32, (N, bs), 0)
    cols = lax.broadcasted_iota(jnp.int32, (N, bs), 1)
    sel = (rows == cols * npb).astype(jnp.float32)
    h0_o[...] = lax.dot_general(sel, out, (((0,), (0,)), ((), ())),
                                preferred_element_type=jnp.float32)

    g = (n2g_r[...] == cols).astype(jnp.float32)
    psum = lax.dot_general(g, out, (((0,), (0,)), ((), ())),
                           preferred_element_type=jnp.float32)
    ones = jnp.ones((N, 1), jnp.float32)
    cnt = lax.dot_general(g, ones, (((0,), (0,)), ((), ())),
                          preferred_element_type=jnp.float32)
    pooled_o[...] = psum / jnp.maximum(cnt, 1.0)


def _finisher(accp, denp, n2g, bgat, bs):
    N = n2g.shape[0]
    return pl.pallas_call(
        _fin_body,
        out_shape=(
            jax.ShapeDtypeStruct((bs, 128), jnp.float32),
            jax.ShapeDtypeStruct((bs, 128), jnp.float32),
        ),
    )(accp, denp, n2g, bgat)


def kernel(qa_emb, x, node_ids, node_types, node_scores, edge_index, edge_type,
           edge_attr, node2graph, W_nt, b_nt, W_x2h, b_x2h, W_e1, b_e1, W_e2,
           b_e2, W_gat, att_src, att_dst, W_ge, att_edge, b_gat):
    bs = qa_emb.shape[0]

    xl, a_src, a_dst, m1, m2, ve = _node_prologue(
        qa_emb, x, node_types, node_scores, W_nt, b_nt.reshape(1, -1),
        W_x2h, b_x2h.reshape(1, -1), W_gat, att_src.reshape(1, -1),
        att_dst.reshape(1, -1), W_ge, att_edge.reshape(1, -1))

    a_edge, m3 = _edge_mlp(edge_attr.T, W_e1, b_e1.reshape(-1, 1), W_e2,
                           b_e2.reshape(-1, 1), ve)

    shift = jnp.full((16,), m1[0, 0] + m2[0, 0] + m3[0, 0], jnp.float32)
    (accp, denp), _npad = _sc_message_passing(
        edge_index.astype(jnp.int32), a_edge.reshape(-1), a_src.reshape(-1),
        a_dst.reshape(-1), xl, shift)

    h0, pooled = _finisher(accp, denp.reshape(-1, 1),
                           node2graph.astype(jnp.int32).reshape(-1, 1),
                           b_gat.reshape(1, -1), bs)
    return (h0, pooled)


# speedup vs baseline: 1.6938x; 1.0169x over previous
import functools

import jax
import jax.numpy as jnp
from jax import lax
from jax.experimental import pallas as pl
from jax.experimental.pallas import tpu as pltpu, tpu_sc as plsc


def _node_body(qa_r, x_r, nt_r, ns_r, Wnt_r, bnt_r, Wx2h_r, bx2h_r, Wgat_r,
               asw_r, adw_r, Wge_r, aew_r,
               xl_o, asrc_o, adst_o, m1_o, m2_o, ve_o):
    N = x_r.shape[0]
    bs = qa_r.shape[0]
    npb = N // bs
    rows = lax.broadcasted_iota(jnp.int32, (N, bs), 0)
    cols = lax.broadcasted_iota(jnp.int32, (N, bs), 1)
    sel = (rows == cols * npb).astype(jnp.float32)
    qa_rows = jnp.dot(sel, qa_r[...], preferred_element_type=jnp.float32)
    is_qa = (rows[:, :1] % npb) == 0
    x2 = jnp.where(is_qa, qa_rows, x_r[...])

    Wnt = Wnt_r[...]
    xe = (jnp.dot(nt_r[...], Wnt[0:4, :], preferred_element_type=jnp.float32)
          + ns_r[...] * Wnt[4:5, :] + bnt_r[...])
    Wx2h = Wx2h_r[...]
    h = jnp.maximum(
        jnp.dot(x2, Wx2h[0:128, :], preferred_element_type=jnp.float32)
        + jnp.dot(xe, Wx2h[128:192, :], preferred_element_type=jnp.float32)
        + bx2h_r[...], 0.0)
    xl = jnp.dot(h, Wgat_r[...], preferred_element_type=jnp.float32)
    xl_o[...] = xl
    asrc = lax.dot_general(asw_r[...], xl, (((1,), (1,)), ((), ())),
                           preferred_element_type=jnp.float32)
    adst = lax.dot_general(adw_r[...], xl, (((1,), (1,)), ((), ())),
                           preferred_element_type=jnp.float32)
    asrc_o[...] = asrc
    adst_o[...] = adst
    m1_o[...] = jnp.max(asrc).reshape(1, 1)
    m2_o[...] = jnp.max(adst).reshape(1, 1)
    ve_o[...] = lax.dot_general(aew_r[...], Wge_r[...],
                                (((1,), (1,)), ((), ())),
                                preferred_element_type=jnp.float32)


def _node_prologue(qa, x, nt, ns, Wnt, bnt, Wx2h, bx2h, Wgat, asw, adw, Wge, aew):
    N = x.shape[0]
    return pl.pallas_call(
        _node_body,
        out_shape=(
            jax.ShapeDtypeStruct((N, 128), jnp.float32),
            jax.ShapeDtypeStruct((1, N), jnp.float32),
            jax.ShapeDtypeStruct((1, N), jnp.float32),
            jax.ShapeDtypeStruct((1, 1), jnp.float32),
            jax.ShapeDtypeStruct((1, 1), jnp.float32),
            jax.ShapeDtypeStruct((1, 128), jnp.float32),
        ),
    )(qa, x, nt, ns, Wnt, bnt, Wx2h, bx2h, Wgat, asw, adw, Wge, aew)


def _edge_body(ea_r, We1_r, be1_r, We2_r, be2_r, ve_r, aedge_o, m3_o, acc):
    i = pl.program_id(0)
    n = pl.num_programs(0)
    t = jnp.maximum(
        lax.dot_general(We1_r[...], ea_r[...], (((0,), (0,)), ((), ())),
                        preferred_element_type=jnp.float32) + be1_r[...], 0.0)
    t = jnp.maximum(
        lax.dot_general(We2_r[...], t, (((0,), (0,)), ((), ())),
                        preferred_element_type=jnp.float32) + be2_r[...], 0.0)
    a = lax.dot_general(ve_r[...], t, (((1,), (0,)), ((), ())),
                        preferred_element_type=jnp.float32)
    aedge_o[0] = a

    @pl.when(i == 0)
    def _():
        acc[0, 0] = -jnp.inf

    acc[0, 0] = jnp.maximum(acc[0, 0], jnp.max(a))

    @pl.when(i == n - 1)
    def _():
        m3_o[...] = jnp.full((1, 1), acc[0, 0], jnp.float32)


def _edge_mlp(edge_attr_t, We1, be1, We2, be2, ve):
    e_in, E = edge_attr_t.shape
    T = 16000
    n = E // T
    return pl.pallas_call(
        _edge_body,
        grid=(n,),
        in_specs=[
            pl.BlockSpec((e_in, T), lambda i: (0, i)),
            pl.BlockSpec((e_in, 128), lambda i: (0, 0)),
            pl.BlockSpec((128, 1), lambda i: (0, 0)),
            pl.BlockSpec((128, 128), lambda i: (0, 0)),
            pl.BlockSpec((128, 1), lambda i: (0, 0)),
            pl.BlockSpec((1, 128), lambda i: (0, 0)),
        ],
        out_specs=(
            pl.BlockSpec((1, 1, T), lambda i: (i, 0, 0)),
            pl.BlockSpec((1, 1), lambda i: (0, 0)),
        ),
        out_shape=(
            jax.ShapeDtypeStruct((n, 1, T), jnp.float32),
            jax.ShapeDtypeStruct((1, 1), jnp.float32),
        ),
        scratch_shapes=[pltpu.SMEM((1, 1), jnp.float32)],
    )(edge_attr_t, We1, be1, We2, be2, ve)


_B = 80
_NSUB = 16
_NW = 32


_NSC = 5


def _sc_body(ei5_r, aef_r, asrc_r, adst_r, xl_r, shift_r, zr_r, zd_r,
             acc_o, den_o,
             srci2, dsti2, aef, asb, adb, exb, rows, shiftv,
             acc_sh, den_sh, sem0, sem1, semd0, semd1):
    E = aef_r.shape[0]
    epw = E // _NW
    nbw = epw // _B
    nbs = nbw // _NSC
    eps = epw // _NSC
    npad = den_sh.shape[0]
    rpt = npad // _NSUB
    cid = lax.axis_index("c")
    sid = lax.axis_index("s")
    wid = cid * _NSUB + sid
    sems = (sem0, sem1)
    semd = (semd0, semd1)

    pltpu.sync_copy(zr_r.at[pl.ds(sid * rpt, rpt)],
                    acc_sh.at[pl.ds(sid * rpt, rpt)])

    @pl.when(sid == 0)
    def _():
        pltpu.sync_copy(zd_r, den_sh)

    pltpu.sync_copy(shift_r, shiftv)
    plsc.subcore_barrier()

    def issue(j, b):
        pltpu.async_copy(xl_r.at[srci2.at[j]], rows.at[b], sems[b])
        pltpu.async_copy(asrc_r.at[srci2.at[j]], asb.at[b], sems[b])
        pltpu.async_copy(adst_r.at[dsti2.at[j]], adb.at[b], sems[b])

    def drain(j, b):
        pltpu.make_async_copy(xl_r.at[srci2.at[j]], rows.at[b],
                              sems[b]).wait()
        pltpu.make_async_copy(asrc_r.at[srci2.at[j]], asb.at[b],
                              sems[b]).wait()
        pltpu.make_async_copy(adst_r.at[dsti2.at[j]], adb.at[b],
                              sems[b]).wait()

    def process(j, b):
        shv = shiftv[...]

        @pl.when(j >= 2)
        def _():
            pltpu.make_async_copy(exb.at[b], den_sh.at[dsti2.at[j - 2]],
                                  semd[b]).wait()

        for i in range(_B // 16):
            al = (asb[b, pl.ds(i * 16, 16)] + adb[b, pl.ds(i * 16, 16)]
                  + aef[pl.ds(j * _B + i * 16, 16)])
            al = jnp.where(al > 0.0, al, al * 0.2)
            exb[b, pl.ds(i * 16, 16)] = jnp.exp(al - shv)

        pltpu.async_copy(exb.at[b], den_sh.at[dsti2.at[j]], semd[b], add=True)

        def rowgrp(i, c):
            ev = exb[b, pl.ds(i * 16, 16)]
            for r in range(16):
                e = ev[r]
                row = i * 16 + r
                for k in range(8):
                    rows[b, row, pl.ds(k * 16, 16)] = (
                        rows[b, row, pl.ds(k * 16, 16)] * e)
            return c

        lax.fori_loop(0, _B // 16, rowgrp, 0, unroll=False)

        pltpu.sync_copy(rows.at[b], acc_sh.at[dsti2.at[j]], add=True)

    def superchunk(s, carry0):
        pltpu.sync_copy(ei5_r.at[0, wid, s], srci2)
        pltpu.sync_copy(ei5_r.at[1, wid, s], dsti2)
        pltpu.sync_copy(aef_r.at[pl.ds(wid * epw + s * eps, eps)], aef)

        issue(0, 0)
        issue(1, 1)

        def pair(g, carry):
            for b in range(2):
                j = 2 * g + b
                drain(j, b)
                process(j, b)

                @pl.when(j + 2 < nbs)
                def _():
                    issue(j + 2, b)
            return carry

        lax.fori_loop(0, (nbs - 1) // 2, pair, 0, unroll=False)
        jt = nbs - 1
        drain(jt, 0)
        process(jt, 0)
        pltpu.make_async_copy(exb.at[0], den_sh.at[dsti2.at[jt]],
                              semd[0]).wait()
        pltpu.make_async_copy(exb.at[1], den_sh.at[dsti2.at[jt - 1]],
                              semd[1]).wait()
        return carry0

    lax.fori_loop(0, _NSC, superchunk, 0, unroll=False)
    plsc.subcore_barrier()

    pltpu.sync_copy(acc_sh.at[pl.ds(sid * rpt, rpt)],
                    acc_o.at[pl.ds(cid * npad + sid * rpt, rpt)])

    @pl.when(sid == 0)
    def _():
        pltpu.sync_copy(den_sh, den_o.at[pl.ds(cid * npad, npad)])


def _sc_message_passing(edge_index, a_edge, a_src, a_dst, xl, shift):
    N = xl.shape[0]
    E = edge_index.shape[1]
    epw = E // _NW
    nbw = epw // _B
    nbs = nbw // _NSC
    npad = ((N + 8 * _NSUB - 1) // (8 * _NSUB)) * (8 * _NSUB)
    mesh = plsc.VectorSubcoreMesh(core_axis_name="c", subcore_axis_name="s")
    ei5 = edge_index.reshape(2, _NW, _NSC, nbs, _B)

    kfn = functools.partial(
        pl.kernel,
        out_type=(
            jax.ShapeDtypeStruct((2 * npad, 128), jnp.float32),
            jax.ShapeDtypeStruct((2 * npad,), jnp.float32),
        ),
        mesh=mesh,
        scratch_types=[
            pltpu.VMEM((nbs, _B), jnp.int32),
            pltpu.VMEM((nbs, _B), jnp.int32),
            pltpu.VMEM((epw // _NSC,), jnp.float32),
            pltpu.VMEM((2, _B), jnp.float32),
            pltpu.VMEM((2, _B), jnp.float32),
            pltpu.VMEM((2, _B), jnp.float32),
            pltpu.VMEM((2, _B, 128), jnp.float32),
            pltpu.VMEM((16,), jnp.float32),
            pltpu.VMEM_SHARED((npad, 128), jnp.float32),
            pltpu.VMEM_SHARED((npad,), jnp.float32),
            pltpu.SemaphoreType.DMA,
            pltpu.SemaphoreType.DMA,
            pltpu.SemaphoreType.DMA,
            pltpu.SemaphoreType.DMA,
        ],
        compiler_params=pltpu.CompilerParams(needs_layout_passes=False),
    )(_sc_body)
    zr = jnp.zeros((npad, 128), jnp.float32)
    zd = jnp.zeros((npad,), jnp.float32)
    accp, denp = kfn(ei5, a_edge, a_src, a_dst, xl, shift, zr, zd)
    return (accp, denp), npad


def _fin_body(accp_r, denp_r, n2g_r, bgat_r, h0_o, pooled_o):
    N = n2g_r.shape[0]
    npad = accp_r.shape[0] // 2
    bs = h0_o.shape[0]
    npb = N // bs
    acc = accp_r[0:N, :] + accp_r[npad:npad + N, :]
    den = denp_r[0:N, :] + denp_r[npad:npad + N, :]
    out = acc / (den + 1e-16) + bgat_r[...]

    rows = lax.broadcasted_iota(jnp.int32, (N, bs), 0)
    cols = lax.broadcasted_iota(jnp.int32, (N, bs), 1)
    sel = (rows == cols * npb).astype(jnp.float32)
    h0_o[...] = lax.dot_general(sel, out, (((0,), (0,)), ((), ())),
                                preferred_element_type=jnp.float32)

    g = (n2g_r[...] == cols).astype(jnp.float32)
    psum = lax.dot_general(g, out, (((0,), (0,)), ((), ())),
                           preferred_element_type=jnp.float32)
    ones = jnp.ones((N, 1), jnp.float32)
    cnt = lax.dot_general(g, ones, (((0,), (0,)), ((), ())),
                          preferred_element_type=jnp.float32)
    pooled_o[...] = psum / jnp.maximum(cnt, 1.0)


def _finisher(accp, denp, n2g, bgat, bs):
    N = n2g.shape[0]
    return pl.pallas_call(
        _fin_body,
        out_shape=(
            jax.ShapeDtypeStruct((bs, 128), jnp.float32),
            jax.ShapeDtypeStruct((bs, 128), jnp.float32),
        ),
    )(accp, denp, n2g, bgat)


def kernel(qa_emb, x, node_ids, node_types, node_scores, edge_index, edge_type,
           edge_attr, node2graph, W_nt, b_nt, W_x2h, b_x2h, W_e1, b_e1, W_e2,
           b_e2, W_gat, att_src, att_dst, W_ge, att_edge, b_gat):
    bs = qa_emb.shape[0]

    xl, a_src, a_dst, m1, m2, ve = _node_prologue(
        qa_emb, x, node_types, node_scores, W_nt, b_nt.reshape(1, -1),
        W_x2h, b_x2h.reshape(1, -1), W_gat, att_src.reshape(1, -1),
        att_dst.reshape(1, -1), W_ge, att_edge.reshape(1, -1))

    a_edge, m3 = _edge_mlp(edge_attr.T, W_e1, b_e1.reshape(-1, 1), W_e2,
                           b_e2.reshape(-1, 1), ve)

    shift = jnp.full((16,), m1[0, 0] + m2[0, 0] + m3[0, 0], jnp.float32)
    (accp, denp), _npad = _sc_message_passing(
        edge_index.astype(jnp.int32), a_edge.reshape(-1), a_src.reshape(-1),
        a_dst.reshape(-1), xl, shift)

    h0, pooled = _finisher(accp, denp.reshape(-1, 1),
                           node2graph.astype(jnp.int32).reshape(-1, 1),
                           b_gat.reshape(1, -1), bs)
    return (h0, pooled)
